# baseline (speedup 1.0000x reference)
import functools

import jax
import jax.numpy as jnp
from jax import lax
from jax.experimental import pallas as pl
from jax.experimental.pallas import tpu as pltpu
from jax.experimental.pallas import tpu_sc as plsc

D = 64
NC, NS = 2, 16
NW = NC * NS
B = 128
NB = 4
DG = 2


@functools.lru_cache(maxsize=None)
def _build(n_batch, n_seq):
    n_units = (n_batch // B) * n_seq
    u_per_w = n_units // NW
    nbt = n_batch // B
    mesh = plsc.VectorSubcoreMesh(core_axis_name="c", subcore_axis_name="s")

    @functools.partial(
        pl.kernel,
        mesh=mesh,
        out_type=jax.ShapeDtypeStruct((n_seq, D // 8, nbt, 8, B), jnp.float32),
        scratch_types=[
            pltpu.VMEM((u_per_w, B), jnp.int32),
            pltpu.VMEM((NB, B, D), jnp.float32),
            pltpu.VMEM((NB, D // 8, 8, 133), jnp.float32),
            pltpu.SemaphoreType.DMA((NB,)),
            pltpu.SemaphoreType.DMA((NB,)),
        ],
        compiler_params=pltpu.CompilerParams(
            use_tc_tiling_on_sc=False, needs_layout_passes=False,
            disable_bounds_checks=True),
    )
    def gather_kernel(idx_hbm, table_hbm, out_hbm, idx_v, rows_v, t_v,
                      gsem, ssem):
        wid = lax.axis_index("s") * NC + lax.axis_index("c")
        ubase = wid * u_per_w
        pltpu.sync_copy(idx_hbm.at[wid], idx_v)

        iota = lax.iota(jnp.int32, 16)
        i0s = [(jnp.full((16,), d0 * 16, jnp.int32) + iota) // 8
               for d0 in range(D // 16)]
        i1s = [lax.rem(jnp.full((16,), d0 * 16, jnp.int32) + iota, 8)
               for d0 in range(D // 16)]

        def g_copy(u, kb):
            return pltpu.make_async_copy(
                table_hbm.at[idx_v.at[u]], rows_v.at[kb], gsem.at[kb])

        def s_copy(u, kt):
            ug = ubase + u
            s_idx = ug // nbt
            bt = lax.rem(ug, nbt)
            return pltpu.make_async_copy(
                t_v.at[kt, :, :, pl.ds(0, B)], out_hbm.at[s_idx, :, bt],
                ssem.at[kt])

        def transpose(kb, kt):
            rows2 = rows_v.at[kb]
            t3 = t_v.at[kt]

            def tb(b16, carry):
                for bi in range(16):
                    b = b16 * 16 + bi
                    bvec = jnp.full((16,), b, jnp.int32)
                    for d0 in range(D // 16):
                        v = rows2[b, pl.ds(d0 * 16, 16)]
                        plsc.store_scatter(t3, [i0s[d0], i1s[d0], bvec], v)
                return carry

            lax.fori_loop(0, B // 16, tb, 0)

        for u in range(DG):
            g_copy(u, u % NB).start()
        for u in range(DG, NB):
            g_copy(u, u % NB).start()
            g_copy(u - DG, (u - DG) % NB).wait()
            transpose((u - DG) % NB, (u - DG) % NB)
            s_copy(u - DG, (u - DG) % NB).start()

        def block(r, carry):
            for k in range(NB):
                u = r * NB + k
                k1 = (k + NB - DG) % NB
                g_copy(u, k).start()
                g_copy(u - DG, k1).wait()
                s_copy(u - 2 * DG, k).wait()
                transpose(k1, k1)
                s_copy(u - DG, k1).start()
            return carry

        lax.fori_loop(1, u_per_w // NB, block, 0)

        for j in range(2 * DG):
            u = u_per_w + j
            k = u % NB
            k1 = (k + NB - DG) % NB
            if j < DG:
                g_copy(u - DG, k1).wait()
            s_copy(u - 2 * DG, k).wait()
            if j < DG:
                transpose(k1, k1)
                s_copy(u - DG, k1).start()

    return gather_kernel


def kernel(word_batch, table):
    b, s = word_batch.shape
    idx = word_batch.astype(jnp.int32).T.reshape(NW, (b * s) // (NW * B), B)
    out5 = _build(b, s)(idx, table)
    return out5.transpose(2, 4, 0, 1, 3).reshape(b, s, D)


# speedup vs baseline: 1.0209x; 1.0209x over previous
import functools

import jax
import jax.numpy as jnp
from jax import lax
from jax.experimental import pallas as pl
from jax.experimental.pallas import tpu as pltpu
from jax.experimental.pallas import tpu_sc as plsc

D = 64
NC, NS = 2, 16
NW = NC * NS
B = 128
NB = 6
DG = 3


@functools.lru_cache(maxsize=None)
def _build(n_batch, n_seq):
    n_units = (n_batch // B) * n_seq
    u_per_w = n_units // NW
    nbt = n_batch // B
    mesh = plsc.VectorSubcoreMesh(core_axis_name="c", subcore_axis_name="s")

    @functools.partial(
        pl.kernel,
        mesh=mesh,
        out_type=jax.ShapeDtypeStruct((n_seq, D // 8, nbt, 8, B), jnp.float32),
        scratch_types=[
            pltpu.VMEM((u_per_w, B), jnp.int32),
            pltpu.VMEM((NB, B, D), jnp.float32),
            pltpu.VMEM((NB, D // 8, 8, 133), jnp.float32),
            pltpu.SemaphoreType.DMA((NB,)),
            pltpu.SemaphoreType.DMA((NB,)),
        ],
        compiler_params=pltpu.CompilerParams(
            use_tc_tiling_on_sc=False, needs_layout_passes=False,
            disable_bounds_checks=True),
    )
    def gather_kernel(idx_hbm, table_hbm, out_hbm, idx_v, rows_v, t_v,
                      gsem, ssem):
        wid = lax.axis_index("s") * NC + lax.axis_index("c")
        ubase = wid * u_per_w
        pltpu.sync_copy(idx_hbm.at[wid], idx_v)

        iota = lax.iota(jnp.int32, 16)
        i0s = [(jnp.full((16,), d0 * 16, jnp.int32) + iota) // 8
               for d0 in range(D // 16)]
        i1s = [lax.rem(jnp.full((16,), d0 * 16, jnp.int32) + iota, 8)
               for d0 in range(D // 16)]

        def g_copy(u, kb):
            return pltpu.make_async_copy(
                table_hbm.at[idx_v.at[u]], rows_v.at[kb], gsem.at[kb])

        def s_copy(u, kt):
            ug = ubase + u
            s_idx = ug // nbt
            bt = lax.rem(ug, nbt)
            return pltpu.make_async_copy(
                t_v.at[kt, :, :, pl.ds(0, B)], out_hbm.at[s_idx, :, bt],
                ssem.at[kt])

        def transpose(kb, kt):
            rows2 = rows_v.at[kb]
            t3 = t_v.at[kt]

            def tb(b8, carry):
                for bi in range(8):
                    b = b8 * 8 + bi
                    bvec = jnp.full((16,), b, jnp.int32)
                    for d0 in range(D // 16):
                        v = rows2[b, pl.ds(d0 * 16, 16)]
                        plsc.store_scatter(t3, [i0s[d0], i1s[d0], bvec], v)
                return carry

            lax.fori_loop(0, B // 8, tb, 0)

        n_blocks = (u_per_w + 2 * DG) // NB + 1

        def block(r, carry):
            for k in range(NB):
                u = r * NB + k
                k1 = (k + NB - DG) % NB

                @pl.when(u < u_per_w)
                def _():
                    g_copy(u, k).start()

                @pl.when(jnp.logical_and(u - DG >= 0, u - DG < u_per_w))
                def _():
                    g_copy(u - DG, k1).wait()

                @pl.when(jnp.logical_and(u - 2 * DG >= 0,
                                         u - 2 * DG < u_per_w))
                def _():
                    s_copy(u - 2 * DG, k).wait()

                @pl.when(jnp.logical_and(u - DG >= 0, u - DG < u_per_w))
                def _():
                    transpose(k1, k1)
                    s_copy(u - DG, k1).start()
            return carry

        lax.fori_loop(0, n_blocks, block, 0)

    return gather_kernel


def kernel(word_batch, table):
    b, s = word_batch.shape
    idx = word_batch.astype(jnp.int32).T.reshape(NW, (b * s) // (NW * B), B)
    out5 = _build(b, s)(idx, table)
    return out5.transpose(2, 4, 0, 1, 3).reshape(b, s, D)
